# baseline (speedup 1.0000x reference)
import functools

import jax
import jax.numpy as jnp
from jax import lax
from jax.experimental import pallas as pl
from jax.experimental.pallas import tpu as pltpu
from jax.experimental.pallas import tpu_sc as plsc

N = 50000
NP = 51200
E = 800000
EP = 819200
V = 10000
G = 128

NCORE = 2
NSUB = 16
NTILE = NCORE * NSUB
EPW = EP // NTILE
GPW = NP // NTILE
GCH = 64

DCH = 1024
DROWS = DCH // 128
DACC = NP + 128

RNG = 10240
RACC = RNG + 128
NPASS = NP // RNG


def _mesh():
    return plsc.VectorSubcoreMesh(core_axis_name="c", subcore_axis_name="s")


def _fill_rows(buf, val16, lanes):
    def fill(i, _):
        r = i // lanes
        l = i - r * lanes
        buf[r, pl.ds(l * 16, 16)] = val16
        return _
    lax.fori_loop(0, 128 * lanes, fill, None)


def _zero_share(zrows, acc, sid, nrows):
    zt = nrows // NSUB
    zfull, ztail = zt // 128, zt % 128
    zbase = sid * zt
    for k in range(zfull):
        pltpu.sync_copy(zrows, acc.at[pl.ds(zbase + k * 128, 128)])
    if ztail:
        pltpu.sync_copy(zrows.at[pl.ds(0, ztail)],
                        acc.at[pl.ds(zbase + zfull * 128, ztail)])


def _drain_share(acc, bounce, out_hbm, sid, nrows, obase):
    dt = nrows // NSUB
    for k in range(dt // 128):
        pltpu.sync_copy(acc.at[pl.ds(sid * dt + k * 128, 128)], bounce)
        pltpu.sync_copy(bounce,
                        out_hbm.at[pl.ds(obase + sid * dt + k * 128, 128)])


def _gather_body(x_hbm, t1_hbm, out_hbm, idxv, rows, sem):
    cid = lax.axis_index("c")
    sid = lax.axis_index("s")
    wid = cid * NSUB + sid
    base = wid * GPW
    pltpu.sync_copy(x_hbm.at[pl.ds(base, GPW)], idxv)

    def ch(j, _):
        pltpu.async_copy(t1_hbm.at[idxv.at[pl.ds(j * GCH, GCH)]], rows, sem).wait()
        pltpu.sync_copy(rows, out_hbm.at[pl.ds(base + j * GCH, GCH)])
        return _
    lax.fori_loop(0, GPW // GCH, ch, None)


_sc_gather = functools.partial(
    pl.kernel, _gather_body,
    out_type=jax.ShapeDtypeStruct((NP, 128), jnp.float32),
    mesh=_mesh(),
    scratch_types=[
        pltpu.VMEM((GPW,), jnp.int32),
        pltpu.VMEM((GCH, 128), jnp.float32),
        pltpu.SemaphoreType.DMA,
    ],
)


def _segsum_body(src_hbm, dst_hbm, y_hbm, out_hbm,
                 ssel, draw, dsel, rows, zrows, sem, acc):
    cid = lax.axis_index("c")
    sid = lax.axis_index("s")
    wid = cid * NSUB + sid
    ebase = wid * EPW
    rbase = wid * (EPW // 128)
    _fill_rows(zrows, jnp.zeros((16,), jnp.float32), 8)

    for p in range(NPASS):
        lo = p * RNG
        _zero_share(zrows, acc, sid, RACC - 128)
        plsc.subcore_barrier()

        def chunk(ci, _):
            pltpu.sync_copy(src_hbm.at[pl.ds(ebase + ci * DCH, DCH)], ssel)
            pltpu.sync_copy(dst_hbm.at[pl.ds(rbase + ci * DROWS, DROWS)], draw)

            def conv(i, _):
                r = i // 8
                l = i - r * 8
                d16 = draw[r, pl.ds(l * 16, 16)]
                moff = d16 - lo
                iv = lax.shift_right_arithmetic(
                    moff | (RNG - 1 - moff), 31) + 1
                dsel[r, pl.ds(l * 16, 16)] = iv * (moff - RNG) + RNG
                return _
            lax.fori_loop(0, DROWS * 8, conv, None)

            def row(j, _):
                pltpu.async_copy(
                    y_hbm.at[ssel.at[pl.ds(j * 128, 128)]], rows, sem).wait()
                pltpu.sync_copy(rows, acc.at[dsel.at[j]], add=True)
                return _
            return lax.fori_loop(0, DROWS, row, _)
        lax.fori_loop(0, EPW // DCH, chunk, None)

        plsc.subcore_barrier()
        _drain_share(acc, rows, out_hbm, sid, RNG, cid * NP + lo)
        plsc.subcore_barrier()


_sc_segsum = functools.partial(
    pl.kernel, _segsum_body,
    out_type=jax.ShapeDtypeStruct((NCORE * NP, 128), jnp.float32),
    mesh=_mesh(),
    scratch_types=[
        pltpu.VMEM((DCH,), jnp.int32),
        pltpu.VMEM((DROWS, 128), jnp.int32),
        pltpu.VMEM((DROWS, 128), jnp.int32),
        pltpu.VMEM((128, 128), jnp.float32),
        pltpu.VMEM((128, 128), jnp.float32),
        pltpu.SemaphoreType.DMA,
        pltpu.VMEM_SHARED((RACC, 128), jnp.float32),
    ],
)


def _dis(p0_ref, p1_ref, pid, nrow):
    deg = jnp.sum(p0_ref[:] + p1_ref[:], axis=1, keepdims=True) * (1.0 / 16.0)
    rows = lax.broadcasted_iota(jnp.int32, (nrow, 1), 0) + pid * nrow
    deg = deg + jnp.where(rows < N, 1.0, 0.0)
    return jnp.where(deg > 0, lax.rsqrt(deg), 0.0)


def _t1_body(emb_ref, w_ref, o_ref):
    o_ref[:] = jnp.dot(emb_ref[:], w_ref[:], preferred_element_type=jnp.float32)


def _scale_body(y_ref, p0_ref, p1_ref, o_ref):
    dis = _dis(p0_ref, p1_ref, pl.program_id(0), y_ref.shape[0])
    o_ref[:] = y_ref[:] * dis


def _combine_body(s0_ref, s1_ref, y_ref, p0_ref, p1_ref, b_ref, w_ref, *outs):
    dis = _dis(p0_ref, p1_ref, pl.program_id(0), y_ref.shape[0])
    h = jnp.maximum(dis * (s0_ref[:] + s1_ref[:] + y_ref[:]) + b_ref[:], 0.0)
    o = dis * jnp.dot(h, w_ref[:], preferred_element_type=jnp.float32)
    for g, o_ref in enumerate(outs):
        o_ref[:] = o[:, g * 128:(g + 1) * 128]


def _final_body(sa0, sa1, sb0, sb1, ya_ref, yb_ref, p0_ref, p1_ref, b_ref,
                bat_ref, pool_ref, cnt_ref):
    pid = pl.program_id(0)
    nrow = ya_ref.shape[0]
    dis = _dis(p0_ref, p1_ref, pid, nrow)
    ha = dis * (sa0[:] + sa1[:] + ya_ref[:]) + b_ref[:, :128]
    hb = dis * (sb0[:] + sb1[:] + yb_ref[:]) + b_ref[:, 128:]
    h = jnp.maximum(jnp.concatenate([ha, hb], axis=1), 0.0)
    oh = (bat_ref[:] == lax.broadcasted_iota(jnp.int32, (nrow, G), 1))
    oh = oh.astype(jnp.float32)

    @pl.when(pid == 0)
    def _():
        pool_ref[:] = jnp.zeros_like(pool_ref)
        cnt_ref[:] = jnp.zeros_like(cnt_ref)

    dn = (((0,), (0,)), ((), ()))
    pool_ref[:] += lax.dot_general(oh, h, dn, preferred_element_type=jnp.float32)
    cnt_ref[:] += lax.dot_general(oh, jnp.ones((nrow, 256), jnp.float32), dn,
                                  preferred_element_type=jnp.float32)


def _mlp_body(pool_ref, cnt_ref, w1_ref, b1_ref, w2_ref, b2_ref, o_ref):
    pooled = pool_ref[:] / jnp.maximum(cnt_ref[:], 1.0)
    h = jnp.maximum(
        jnp.dot(pooled, w1_ref[:], preferred_element_type=jnp.float32) + b1_ref[:],
        0.0)
    o = jnp.dot(h, w2_ref[:], preferred_element_type=jnp.float32) + b2_ref[:]
    o_ref[:] = 1.0 / (1.0 + jnp.exp(-o))


_BR = 1024
_GRID = NP // _BR


def _row_spec(dd):
    return pl.BlockSpec((_BR, dd), lambda i: (i, 0))


def _const_spec(shape):
    return pl.BlockSpec(shape, lambda i: (0, 0))


def _k_scale(y, p0, p1):
    return pl.pallas_call(
        _scale_body,
        grid=(_GRID,),
        in_specs=[_row_spec(128), _row_spec(16), _row_spec(16)],
        out_specs=_row_spec(128),
        out_shape=jax.ShapeDtypeStruct((NP, 128), jnp.float32),
    )(y, p0, p1)


def _k_combine(s, y, p0, p1, b, w):
    dout = w.shape[1]
    ngout = dout // 128
    outs = pl.pallas_call(
        _combine_body,
        grid=(_GRID,),
        in_specs=[_row_spec(128), _row_spec(128), _row_spec(128),
                  _row_spec(16), _row_spec(16),
                  _const_spec((1, 128)), _const_spec((128, dout))],
        out_specs=[_row_spec(128)] * ngout,
        out_shape=[jax.ShapeDtypeStruct((NP, 128), jnp.float32)] * ngout,
    )(s[:NP], s[NP:], y, p0, p1, b, w)
    return outs


def _k_final(sa, sb, ya, yb, p0, p1, b, bat):
    return pl.pallas_call(
        _final_body,
        grid=(_GRID,),
        in_specs=[_row_spec(128)] * 4 + [_row_spec(128)] * 2
        + [_row_spec(16), _row_spec(16),
           _const_spec((1, 256)), _row_spec(1)],
        out_specs=[_const_spec((G, 256)), _const_spec((G, 256))],
        out_shape=[jax.ShapeDtypeStruct((G, 256), jnp.float32),
                   jax.ShapeDtypeStruct((G, 256), jnp.float32)],
    )(sa[:NP], sa[NP:], sb[:NP], sb[NP:], ya, yb, p0, p1, b, bat)


def kernel(x, edge_index, batch, emb_table, W1, b1, W2, b2, W3, b3,
           L1W, L1b, L2W, L2b):
    srcp = jnp.pad(edge_index[0].astype(jnp.int32), (0, EP - E))
    dstp = jnp.pad(edge_index[1].astype(jnp.int32), (0, EP - E),
                   constant_values=NP)
    dst2 = dstp.reshape(EP // 128, 128)
    xp = jnp.pad(x.astype(jnp.int32), (0, NP - N))
    batp = jnp.pad(batch.astype(jnp.int32), (0, NP - N),
                   constant_values=-1).reshape(NP, 1)
    embp = jnp.pad(emb_table, ((0, 0), (0, 128 - 80)))
    w1p = jnp.pad(W1, ((0, 128 - 80), (0, 0)))

    ones_y = jnp.ones((NP, 128), jnp.float32)
    pS = _sc_segsum()(srcp, dst2, ones_y)
    p0, p1 = pS[:NP, :16], pS[NP:, :16]

    t1 = pl.pallas_call(
        _t1_body,
        out_shape=jax.ShapeDtypeStruct((V, 128), jnp.float32),
    )(embp, w1p)

    y1p = _sc_gather()(xp, t1)
    y1 = _k_scale(y1p, p0, p1)
    s1 = _sc_segsum()(srcp, dst2, y1)
    (y2,) = _k_combine(s1, y1, p0, p1, b1.reshape(1, 128), W2)
    s2 = _sc_segsum()(srcp, dst2, y2)
    y3a, y3b = _k_combine(s2, y2, p0, p1, b2.reshape(1, 128), W3)
    s3a = _sc_segsum()(srcp, dst2, y3a)
    s3b = _sc_segsum()(srcp, dst2, y3b)
    pool, cnt = _k_final(s3a, s3b, y3a, y3b, p0, p1,
                         b3.reshape(1, 256), batp)

    out = pl.pallas_call(
        _mlp_body,
        out_shape=jax.ShapeDtypeStruct((G, L2W.shape[1]), jnp.float32),
    )(pool, cnt, L1W, L1b.reshape(1, 512), L2W, L2b.reshape(1, 128))
    return out


# speedup vs baseline: 1.2053x; 1.2053x over previous
import functools

import jax
import jax.numpy as jnp
from jax import lax
from jax.experimental import pallas as pl
from jax.experimental.pallas import tpu as pltpu
from jax.experimental.pallas import tpu_sc as plsc

N = 50000
NP = 51200
E = 800000
EP = 819200
V = 10000
G = 128

NCORE = 2
NSUB = 16
NTILE = NCORE * NSUB
EPW = EP // NTILE
GPW = NP // NTILE
GCH = 64

DCH = 1024
DROWS = DCH // 128
DACC = NP + 128

RNG = 10240
RACC = RNG + 128
NPASS = NP // RNG


def _mesh():
    return plsc.VectorSubcoreMesh(core_axis_name="c", subcore_axis_name="s")


def _fill_rows(buf, val16, lanes):
    def fill(i, _):
        r = i // lanes
        l = i - r * lanes
        buf[r, pl.ds(l * 16, 16)] = val16
        return _
    lax.fori_loop(0, 128 * lanes, fill, None)


def _zero_share(zrows, acc, sid, nrows):
    zt = nrows // NSUB
    zfull, ztail = zt // 128, zt % 128
    zbase = sid * zt
    for k in range(zfull):
        pltpu.sync_copy(zrows, acc.at[pl.ds(zbase + k * 128, 128)])
    if ztail:
        pltpu.sync_copy(zrows.at[pl.ds(0, ztail)],
                        acc.at[pl.ds(zbase + zfull * 128, ztail)])


def _drain_share(acc, bounce, out_hbm, sid, nrows, obase):
    dt = nrows // NSUB
    for k in range(dt // 128):
        pltpu.sync_copy(acc.at[pl.ds(sid * dt + k * 128, 128)], bounce)
        pltpu.sync_copy(bounce,
                        out_hbm.at[pl.ds(obase + sid * dt + k * 128, 128)])


def _gather_body(x_hbm, t1_hbm, out_hbm, idxv, rows, sem):
    cid = lax.axis_index("c")
    sid = lax.axis_index("s")
    wid = cid * NSUB + sid
    base = wid * GPW
    pltpu.sync_copy(x_hbm.at[pl.ds(base, GPW)], idxv)

    def ch(j, _):
        pltpu.async_copy(t1_hbm.at[idxv.at[pl.ds(j * GCH, GCH)]], rows, sem).wait()
        pltpu.sync_copy(rows, out_hbm.at[pl.ds(base + j * GCH, GCH)])
        return _
    lax.fori_loop(0, GPW // GCH, ch, None)


_sc_gather = functools.partial(
    pl.kernel, _gather_body,
    out_type=jax.ShapeDtypeStruct((NP, 128), jnp.float32),
    mesh=_mesh(),
    scratch_types=[
        pltpu.VMEM((GPW,), jnp.int32),
        pltpu.VMEM((GCH, 128), jnp.float32),
        pltpu.SemaphoreType.DMA,
    ],
)


def _make_segsum_body(with_gather):
  def _segsum_body(src_hbm, dst_hbm, *args):
    if with_gather:
        y_hbm, out_hbm, ssel, draw, dsel, rows, zrows, sem, acc = args
    else:
        out_hbm, draw, dsel, rows, zrows, acc = args
    cid = lax.axis_index("c")
    sid = lax.axis_index("s")
    wid = cid * NSUB + sid
    ebase = wid * EPW
    rbase = wid * (EPW // 128)
    _fill_rows(zrows, jnp.zeros((16,), jnp.float32), 8)
    if not with_gather:
        _fill_rows(rows, jnp.ones((16,), jnp.float32), 8)

    for p in range(NPASS):
        lo = p * RNG
        _zero_share(zrows, acc, sid, RACC - 128)
        plsc.subcore_barrier()

        def chunk(ci, _):
            if with_gather:
                pltpu.sync_copy(src_hbm.at[pl.ds(ebase + ci * DCH, DCH)], ssel)
            pltpu.sync_copy(dst_hbm.at[pl.ds(rbase + ci * DROWS, DROWS)], draw)

            def conv(i, _):
                r = i // 8
                l = i - r * 8
                d16 = draw[r, pl.ds(l * 16, 16)]
                moff = d16 - lo
                iv = lax.shift_right_arithmetic(
                    moff | (RNG - 1 - moff), 31) + 1
                dsel[r, pl.ds(l * 16, 16)] = iv * (moff - RNG) + RNG
                return _
            lax.fori_loop(0, DROWS * 8, conv, None)

            def row(j, _):
                if with_gather:
                    pltpu.async_copy(
                        y_hbm.at[ssel.at[pl.ds(j * 128, 128)]], rows,
                        sem).wait()
                pltpu.sync_copy(rows, acc.at[dsel.at[j]], add=True)
                return _
            return lax.fori_loop(0, DROWS, row, _)
        lax.fori_loop(0, EPW // DCH, chunk, None)

        plsc.subcore_barrier()
        _drain_share(acc, rows if with_gather else zrows,
                     out_hbm, sid, RNG, cid * NP + lo)
        plsc.subcore_barrier()
        if not with_gather:
            _fill_rows(zrows, jnp.zeros((16,), jnp.float32), 8)
            plsc.subcore_barrier()
  return _segsum_body


_sc_segsum = functools.partial(
    pl.kernel, _make_segsum_body(True),
    out_type=jax.ShapeDtypeStruct((NCORE * NP, 128), jnp.float32),
    mesh=_mesh(),
    scratch_types=[
        pltpu.VMEM((DCH,), jnp.int32),
        pltpu.VMEM((DROWS, 128), jnp.int32),
        pltpu.VMEM((DROWS, 128), jnp.int32),
        pltpu.VMEM((128, 128), jnp.float32),
        pltpu.VMEM((128, 128), jnp.float32),
        pltpu.SemaphoreType.DMA,
        pltpu.VMEM_SHARED((RACC, 128), jnp.float32),
    ],
)

_sc_deg = functools.partial(
    pl.kernel, _make_segsum_body(False),
    out_type=jax.ShapeDtypeStruct((NCORE * NP, 128), jnp.float32),
    mesh=_mesh(),
    scratch_types=[
        pltpu.VMEM((DROWS, 128), jnp.int32),
        pltpu.VMEM((DROWS, 128), jnp.int32),
        pltpu.VMEM((128, 128), jnp.float32),
        pltpu.VMEM((128, 128), jnp.float32),
        pltpu.VMEM_SHARED((RACC, 128), jnp.float32),
    ],
)


def _dis(p0_ref, p1_ref, pid, nrow):
    deg = jnp.sum(p0_ref[:] + p1_ref[:], axis=1, keepdims=True) * (1.0 / 16.0)
    rows = lax.broadcasted_iota(jnp.int32, (nrow, 1), 0) + pid * nrow
    deg = deg + jnp.where(rows < N, 1.0, 0.0)
    return jnp.where(deg > 0, lax.rsqrt(deg), 0.0)


def _t1_body(emb_ref, w_ref, o_ref):
    o_ref[:] = jnp.dot(emb_ref[:], w_ref[:], preferred_element_type=jnp.float32)


def _scale_body(y_ref, p0_ref, p1_ref, o_ref):
    dis = _dis(p0_ref, p1_ref, pl.program_id(0), y_ref.shape[0])
    o_ref[:] = y_ref[:] * dis


def _combine_body(s0_ref, s1_ref, y_ref, p0_ref, p1_ref, b_ref, w_ref, *outs):
    dis = _dis(p0_ref, p1_ref, pl.program_id(0), y_ref.shape[0])
    h = jnp.maximum(dis * (s0_ref[:] + s1_ref[:] + y_ref[:]) + b_ref[:], 0.0)
    o = dis * jnp.dot(h, w_ref[:], preferred_element_type=jnp.float32)
    for g, o_ref in enumerate(outs):
        o_ref[:] = o[:, g * 128:(g + 1) * 128]


def _final_body(sa0, sa1, sb0, sb1, ya_ref, yb_ref, p0_ref, p1_ref, b_ref,
                bat_ref, pool_ref, cnt_ref):
    pid = pl.program_id(0)
    nrow = ya_ref.shape[0]
    dis = _dis(p0_ref, p1_ref, pid, nrow)
    ha = dis * (sa0[:] + sa1[:] + ya_ref[:]) + b_ref[:, :128]
    hb = dis * (sb0[:] + sb1[:] + yb_ref[:]) + b_ref[:, 128:]
    h = jnp.maximum(jnp.concatenate([ha, hb], axis=1), 0.0)
    oh = (bat_ref[:] == lax.broadcasted_iota(jnp.int32, (nrow, G), 1))
    oh = oh.astype(jnp.float32)

    @pl.when(pid == 0)
    def _():
        pool_ref[:] = jnp.zeros_like(pool_ref)
        cnt_ref[:] = jnp.zeros_like(cnt_ref)

    dn = (((0,), (0,)), ((), ()))
    pool_ref[:] += lax.dot_general(oh, h, dn, preferred_element_type=jnp.float32)
    cnt_ref[:] += lax.dot_general(oh, jnp.ones((nrow, 256), jnp.float32), dn,
                                  preferred_element_type=jnp.float32)


def _mlp_body(pool_ref, cnt_ref, w1_ref, b1_ref, w2_ref, b2_ref, o_ref):
    pooled = pool_ref[:] / jnp.maximum(cnt_ref[:], 1.0)
    h = jnp.maximum(
        jnp.dot(pooled, w1_ref[:], preferred_element_type=jnp.float32) + b1_ref[:],
        0.0)
    o = jnp.dot(h, w2_ref[:], preferred_element_type=jnp.float32) + b2_ref[:]
    o_ref[:] = 1.0 / (1.0 + jnp.exp(-o))


_BR = 1024
_GRID = NP // _BR


def _row_spec(dd):
    return pl.BlockSpec((_BR, dd), lambda i: (i, 0))


def _const_spec(shape):
    return pl.BlockSpec(shape, lambda i: (0, 0))


def _k_scale(y, p0, p1):
    return pl.pallas_call(
        _scale_body,
        grid=(_GRID,),
        in_specs=[_row_spec(128), _row_spec(16), _row_spec(16)],
        out_specs=_row_spec(128),
        out_shape=jax.ShapeDtypeStruct((NP, 128), jnp.float32),
    )(y, p0, p1)


def _k_combine(s, y, p0, p1, b, w):
    dout = w.shape[1]
    ngout = dout // 128
    outs = pl.pallas_call(
        _combine_body,
        grid=(_GRID,),
        in_specs=[_row_spec(128), _row_spec(128), _row_spec(128),
                  _row_spec(16), _row_spec(16),
                  _const_spec((1, 128)), _const_spec((128, dout))],
        out_specs=[_row_spec(128)] * ngout,
        out_shape=[jax.ShapeDtypeStruct((NP, 128), jnp.float32)] * ngout,
    )(s[:NP], s[NP:], y, p0, p1, b, w)
    return outs


def _k_final(sa, sb, ya, yb, p0, p1, b, bat):
    return pl.pallas_call(
        _final_body,
        grid=(_GRID,),
        in_specs=[_row_spec(128)] * 4 + [_row_spec(128)] * 2
        + [_row_spec(16), _row_spec(16),
           _const_spec((1, 256)), _row_spec(1)],
        out_specs=[_const_spec((G, 256)), _const_spec((G, 256))],
        out_shape=[jax.ShapeDtypeStruct((G, 256), jnp.float32),
                   jax.ShapeDtypeStruct((G, 256), jnp.float32)],
    )(sa[:NP], sa[NP:], sb[:NP], sb[NP:], ya, yb, p0, p1, b, bat)


def kernel(x, edge_index, batch, emb_table, W1, b1, W2, b2, W3, b3,
           L1W, L1b, L2W, L2b):
    srcp = jnp.pad(edge_index[0].astype(jnp.int32), (0, EP - E))
    dstp = jnp.pad(edge_index[1].astype(jnp.int32), (0, EP - E),
                   constant_values=NP)
    dst2 = dstp.reshape(EP // 128, 128)
    xp = jnp.pad(x.astype(jnp.int32), (0, NP - N))
    batp = jnp.pad(batch.astype(jnp.int32), (0, NP - N),
                   constant_values=-1).reshape(NP, 1)
    embp = jnp.pad(emb_table, ((0, 0), (0, 128 - 80)))
    w1p = jnp.pad(W1, ((0, 128 - 80), (0, 0)))

    pS = _sc_deg()(srcp, dst2)
    p0, p1 = pS[:NP, :16], pS[NP:, :16]

    t1 = pl.pallas_call(
        _t1_body,
        out_shape=jax.ShapeDtypeStruct((V, 128), jnp.float32),
    )(embp, w1p)

    y1p = _sc_gather()(xp, t1)
    y1 = _k_scale(y1p, p0, p1)
    s1 = _sc_segsum()(srcp, dst2, y1)
    (y2,) = _k_combine(s1, y1, p0, p1, b1.reshape(1, 128), W2)
    s2 = _sc_segsum()(srcp, dst2, y2)
    y3a, y3b = _k_combine(s2, y2, p0, p1, b2.reshape(1, 128), W3)
    s3a = _sc_segsum()(srcp, dst2, y3a)
    s3b = _sc_segsum()(srcp, dst2, y3b)
    pool, cnt = _k_final(s3a, s3b, y3a, y3b, p0, p1,
                         b3.reshape(1, 256), batp)

    out = pl.pallas_call(
        _mlp_body,
        out_shape=jax.ShapeDtypeStruct((G, L2W.shape[1]), jnp.float32),
    )(pool, cnt, L1W, L1b.reshape(1, 512), L2W, L2b.reshape(1, 128))
    return out
